# baseline (speedup 1.0000x reference)
import functools

import jax
import jax.numpy as jnp
from jax import lax
from jax.experimental import pallas as pl
from jax.experimental.pallas import tpu as pltpu
from jax.experimental.pallas import tpu_sc as plsc

_NC = 2
_NS = 16
_NW = _NC * _NS
_L = 16

_GW = 100


def _build(n_rows, seqlen, d_model):
  assert n_rows % (_NW * seqlen) == 0
  rows_w = n_rows // _NW
  seqs_chunk = 4
  rows_chunk = seqs_chunk * seqlen
  n_chunks = rows_w // rows_chunk
  n_gathers = rows_chunk // _GW
  vpr = d_model // _L

  mesh = plsc.VectorSubcoreMesh(
      core_axis_name="c", subcore_axis_name="s",
      num_cores=_NC, num_subcores=_NS)

  @functools.partial(
      pl.kernel,
      out_type=(
          jax.ShapeDtypeStruct((n_rows, d_model), jnp.float32),
          jax.ShapeDtypeStruct((n_rows,), jnp.float32),
      ),
      mesh=mesh,
      scratch_types=(
          [pltpu.VMEM((_GW,), jnp.int32)] * n_gathers +
          [
              pltpu.VMEM((rows_chunk,), jnp.int32),
              pltpu.VMEM((rows_chunk, d_model), jnp.float32),
              pltpu.VMEM((seqlen, d_model), jnp.float32),
              pltpu.VMEM((rows_chunk,), jnp.float32),
              pltpu.SemaphoreType.DMA,
          ]
      ),
      compiler_params=pltpu.CompilerParams(use_tc_tiling_on_sc=False),
  )
  def embed(idx2_hbm, idxf_hbm, tab_hbm, pos_hbm, out_hbm, mask_hbm, *scratch):
    idx_gs = scratch[:n_gathers]
    idxf_v, rows_v, pos_v, mask_v, sem = scratch[n_gathers:]
    wid = lax.axis_index("s") * _NC + lax.axis_index("c")
    base = wid * rows_w
    pltpu.sync_copy(pos_hbm, pos_v)

    def chunk_body(c, carry):
      rb = base + c * rows_chunk
      rb2 = rb // _GW
      pltpu.sync_copy(idxf_hbm.at[pl.ds(rb, rows_chunk)], idxf_v)
      for g in range(n_gathers):
        pltpu.sync_copy(idx2_hbm.at[rb2 + g], idx_gs[g])
      copies = [
          pltpu.async_copy(tab_hbm.at[idx_gs[g]],
                           rows_v.at[pl.ds(g * _GW, _GW)], sem)
          for g in range(n_gathers)
      ]
      def mask_body(m, mc):
        ids = idxf_v[pl.ds(m * _L, _L)]
        mask_v[pl.ds(m * _L, _L)] = jnp.where(
            ids == 0, jnp.float32(-jnp.inf), jnp.float32(0.0))
        return mc
      lax.fori_loop(0, rows_chunk // _L, mask_body, 0)
      for cp in copies:
        cp.wait()

      def pos_body(p, pc):
        for v in range(vpr):
          sl = pl.ds(v * _L, _L)
          pv = pos_v[p, sl]
          for s in range(seqs_chunk):
            r = s * seqlen + p
            rows_v[r, sl] = rows_v[r, sl] + pv
        return pc
      lax.fori_loop(0, seqlen, pos_body, 0)

      pltpu.sync_copy(rows_v, out_hbm.at[pl.ds(rb, rows_chunk)])
      pltpu.sync_copy(mask_v, mask_hbm.at[pl.ds(rb, rows_chunk)])
      return carry

    lax.fori_loop(0, n_chunks, chunk_body, 0)

  return embed


def kernel(tgt, embedding_tgt, embedding_pos):
  batch, seqlen = tgt.shape
  d_model = embedding_tgt.shape[1]
  n_rows = batch * seqlen
  idx_flat = tgt.reshape(n_rows)
  idx2 = idx_flat.reshape(n_rows // _GW, _GW)
  embed = _build(n_rows, seqlen, d_model)
  out, mask = embed(idx2, idx_flat, embedding_tgt, embedding_pos)
  return out.reshape(batch, seqlen, d_model), mask.reshape(batch, seqlen)
